# Initial kernel scaffold; baseline (speedup 1.0000x reference)
import jax
import jax.numpy as jnp
from jax.experimental import pallas as pl


def kernel(x, edge_index, edge_inform, W, att_l, bias):
    raise NotImplementedError("write your pallas kernel here")


# speedup vs baseline: 6.0546x; 6.0546x over previous
import functools

import jax
import jax.numpy as jnp
from jax import lax
from jax.experimental import pallas as pl
from jax.experimental.pallas import tpu as pltpu
from jax.experimental.pallas import tpu_sc as plsc

NC = 2
NS = 16
NW = NC * NS
CH = 128


def _t1a(x, WxT, att2):
    N, D = x.shape
    BN = 1000

    def body(x_ref, w_ref, a_ref, xw_ref, an_ref):
        xw = jnp.dot(x_ref[...], w_ref[...], preferred_element_type=jnp.float32)
        xw_ref[...] = xw
        an_ref[...] = jnp.dot(xw, a_ref[...], preferred_element_type=jnp.float32)

    return pl.pallas_call(
        body,
        grid=(N // BN,),
        in_specs=[
            pl.BlockSpec((BN, D), lambda i: (i, 0)),
            pl.BlockSpec((D, D), lambda i: (0, 0)),
            pl.BlockSpec((D, 1), lambda i: (0, 0)),
        ],
        out_specs=[
            pl.BlockSpec((BN, D), lambda i: (i, 0)),
            pl.BlockSpec((BN, 1), lambda i: (i, 0)),
        ],
        out_shape=[
            jax.ShapeDtypeStruct((N, D), jnp.float32),
            jax.ShapeDtypeStruct((N, 1), jnp.float32),
        ],
    )(x, WxT, att2)


def _t1b(einf, wea2):
    E, DE = einf.shape
    BE = 6400

    def body(e_ref, w_ref, o_ref):
        o_ref[...] = jnp.dot(e_ref[...], w_ref[...], preferred_element_type=jnp.float32)

    return pl.pallas_call(
        body,
        grid=(E // BE,),
        in_specs=[
            pl.BlockSpec((BE, DE), lambda i: (i, 0)),
            pl.BlockSpec((DE, 1), lambda i: (0, 0)),
        ],
        out_specs=pl.BlockSpec((BE, 1), lambda i: (i, 0)),
        out_shape=jax.ShapeDtypeStruct((E, 1), jnp.float32),
    )(einf, wea2)


def _t2(out_p, e_p, WeT, bias2, N):
    _, NPo, D = out_p.shape
    DE = WeT.shape[0]
    BN = 1000

    def body(op_ref, ep_ref, w_ref, b_ref, o_ref):
        s = op_ref[0] + op_ref[1]
        ef = ep_ref[0] + ep_ref[1]
        e16 = ef[:, 0:DE]
        denom = ef[:, DE:DE + 1]
        num = s + jnp.dot(e16, w_ref[...], preferred_element_type=jnp.float32)
        o_ref[...] = num / (denom + 1e-16) + b_ref[...]

    return pl.pallas_call(
        body,
        grid=(N // BN,),
        in_specs=[
            pl.BlockSpec((2, BN, D), lambda i: (0, i, 0)),
            pl.BlockSpec((2, BN, 32), lambda i: (0, i, 0)),
            pl.BlockSpec((DE, D), lambda i: (0, 0)),
            pl.BlockSpec((1, D), lambda i: (0, 0)),
        ],
        out_specs=pl.BlockSpec((BN, D), lambda i: (i, 0)),
        out_shape=jax.ShapeDtypeStruct((N, D), jnp.float32),
    )(out_p, e_p, WeT, bias2)


def _sc_main(a_node, a_edge, src, dst, xW):
    N = a_node.shape[0]
    D = xW.shape[1]
    E_pad = a_edge.shape[0]
    EPW = E_pad // NW
    NCHK = EPW // CH
    RPT = ((N // NS) + 7) // 8 * 8
    LAST = N - (NS - 1) * RPT
    assert LAST % 8 == 0 and 0 < LAST <= RPT and N % 8 == 0

    mesh = plsc.VectorSubcoreMesh(core_axis_name="c", subcore_axis_name="s",
                                  num_cores=NC, num_subcores=NS)

    @functools.partial(
        pl.kernel,
        out_type=jax.ShapeDtypeStruct((NC, N, D), jnp.float32),
        mesh=mesh,
        compiler_params=pltpu.CompilerParams(needs_layout_passes=False),
        scratch_types=[
            pltpu.VMEM((N,), jnp.float32),
            pltpu.VMEM((CH,), jnp.int32),
            pltpu.VMEM((CH,), jnp.int32),
            pltpu.VMEM((CH,), jnp.float32),
            pltpu.VMEM((CH, D), jnp.float32),
            pltpu.VMEM((CH,), jnp.float32),
            pltpu.VMEM_SHARED((N, D), jnp.float32),
            pltpu.SemaphoreType.DMA,
        ],
    )
    def sc_kernel(an_hbm, ae_hbm, src_hbm, dst_hbm, xw_hbm, outp_hbm,
                  an_v, src_v, dst_v, ae_v, rows_v, ex_v, out_acc, sem):
        cid = lax.axis_index("c")
        sid = lax.axis_index("s")
        wid = sid * NC + cid
        notlast = sid != NS - 1

        zero16 = jnp.zeros((16,), jnp.float32)

        def zrow(r, carry):
            for j in range(D // 16):
                rows_v[r, pl.ds(j * 16, 16)] = zero16
            return carry

        lax.fori_loop(0, CH, zrow, 0)
        rbase = sid * RPT
        for k in range(LAST // CH):
            pltpu.sync_copy(rows_v, out_acc.at[pl.ds(rbase + k * CH, CH)])

        @pl.when(notlast)
        def _():
            for k in range(LAST // CH, RPT // CH):
                pltpu.sync_copy(rows_v, out_acc.at[pl.ds(rbase + k * CH, CH)])
            if RPT % CH:
                pltpu.sync_copy(rows_v.at[pl.ds(0, RPT % CH)],
                                out_acc.at[pl.ds(rbase + (RPT // CH) * CH,
                                                 RPT % CH)])

        if LAST % CH:
            @pl.when(jnp.logical_not(notlast))
            def _():
                pltpu.sync_copy(rows_v.at[pl.ds(0, LAST % CH)],
                                out_acc.at[pl.ds(rbase + (LAST // CH) * CH,
                                                 LAST % CH)])

        pltpu.sync_copy(an_hbm, an_v)
        plsc.subcore_barrier()

        def chunk(ci, carry):
            base = wid * EPW + ci * CH
            pltpu.sync_copy(src_hbm.at[pl.ds(base, CH)], src_v)
            pltpu.sync_copy(dst_hbm.at[pl.ds(base, CH)], dst_v)
            pltpu.sync_copy(ae_hbm.at[pl.ds(base, CH)], ae_v)
            pltpu.async_copy(xw_hbm.at[src_v], rows_v, sem).wait()

            def grp(g, c):
                s16 = src_v[pl.ds(g * 16, 16)]
                an = plsc.load_gather(an_v, [s16])
                al = an + ae_v[pl.ds(g * 16, 16)]
                al = jnp.maximum(al, al * 0.01)
                ex_v[pl.ds(g * 16, 16)] = jnp.exp(al)
                return c

            lax.fori_loop(0, CH // 16, grp, 0)

            def scale(g, c):
                ex16 = ex_v[pl.ds(g * 16, 16)]
                for l in range(16):
                    exs = ex16[l]
                    i = g * 16 + l
                    for j in range(D // 16):
                        sl = pl.ds(j * 16, 16)
                        rows_v[i, sl] = rows_v[i, sl] * exs
                return c

            lax.fori_loop(0, CH // 16, scale, 0)
            pltpu.sync_copy(rows_v, out_acc.at[dst_v], add=True)
            return carry

        lax.fori_loop(0, NCHK, chunk, 0)
        plsc.subcore_barrier()

        @pl.when(notlast)
        def _():
            pltpu.sync_copy(out_acc.at[pl.ds(rbase, RPT)],
                            outp_hbm.at[cid, pl.ds(rbase, RPT)])

        @pl.when(jnp.logical_not(notlast))
        def _():
            pltpu.sync_copy(out_acc.at[pl.ds(rbase, LAST)],
                            outp_hbm.at[cid, pl.ds(rbase, LAST)])

    return sc_kernel(a_node, a_edge, src, dst, xW)


def _sc_edge(a_node, a_edge, src, dst, einf_flat, DE):
    N = a_node.shape[0]
    E_pad = a_edge.shape[0]
    EPW = E_pad // NW
    NCHK = EPW // CH
    NPe = ((N + NS * 32 - 1) // (NS * 32)) * (NS * 32)
    NP4 = NPe // 4
    RPT = NP4 // NS

    mesh = plsc.VectorSubcoreMesh(core_axis_name="c", subcore_axis_name="s",
                                  num_cores=NC, num_subcores=NS)

    @functools.partial(
        pl.kernel,
        out_type=jax.ShapeDtypeStruct((NC, NP4, 128), jnp.float32),
        mesh=mesh,
        compiler_params=pltpu.CompilerParams(needs_layout_passes=False),
        scratch_types=[
            pltpu.VMEM((N,), jnp.float32),
            pltpu.VMEM((CH,), jnp.int32),
            pltpu.VMEM((CH,), jnp.int32),
            pltpu.VMEM((CH,), jnp.int32),
            pltpu.VMEM((CH,), jnp.float32),
            pltpu.VMEM((CH * 16,), jnp.float32),
            pltpu.VMEM((CH, 128), jnp.float32),
            pltpu.VMEM_SHARED((NP4, 128), jnp.float32),
        ],
    )
    def sc_kernel(an_hbm, ae_hbm, src_hbm, dst_hbm, einf_hbm, ep_hbm,
                  an_v, src_v, dst_v, ridx_v, ae_v, einf_v, val_v, e_acc):
        cid = lax.axis_index("c")
        sid = lax.axis_index("s")
        wid = sid * NC + cid

        zero16 = jnp.zeros((16,), jnp.float32)

        def zrow(r, carry):
            for j in range(8):
                val_v[r, pl.ds(j * 16, 16)] = zero16
            return carry

        lax.fori_loop(0, CH, zrow, 0)
        rbase = sid * RPT
        for k in range(RPT // CH):
            pltpu.sync_copy(val_v, e_acc.at[pl.ds(rbase + k * CH, CH)])
        zrem = RPT - (RPT // CH) * CH
        if zrem:
            pltpu.sync_copy(val_v.at[pl.ds(0, zrem)],
                            e_acc.at[pl.ds(rbase + (RPT // CH) * CH, zrem)])
        pltpu.sync_copy(an_hbm, an_v)
        plsc.subcore_barrier()

        lane0 = lax.iota(jnp.int32, 16) == 0

        def chunk(ci, carry):
            base = wid * EPW + ci * CH
            pltpu.sync_copy(src_hbm.at[pl.ds(base, CH)], src_v)
            pltpu.sync_copy(dst_hbm.at[pl.ds(base, CH)], dst_v)
            pltpu.sync_copy(ae_hbm.at[pl.ds(base, CH)], ae_v)
            pltpu.sync_copy(einf_hbm.at[pl.ds(base * DE, CH * DE)], einf_v)

            def grp(g, c):
                s16 = src_v[pl.ds(g * 16, 16)]
                d16 = dst_v[pl.ds(g * 16, 16)]
                ridx_v[pl.ds(g * 16, 16)] = lax.shift_right_logical(d16, 2)
                dmod = lax.bitwise_and(d16, 3)
                an = plsc.load_gather(an_v, [s16])
                al = an + ae_v[pl.ds(g * 16, 16)]
                al = jnp.maximum(al, al * 0.01)
                ex16 = jnp.exp(al)
                for l in range(16):
                    exs = ex16[l]
                    q = dmod[l]
                    i = g * 16 + l
                    for j in range(8):
                        val_v[i, pl.ds(j * 16, 16)] = zero16
                    val_v[i, pl.ds(q * 32, 16)] = \
                        einf_v[pl.ds(i * DE, 16)] * exs
                    val_v[i, pl.ds(q * 32 + 16, 16)] = jnp.where(lane0, exs, 0.0)
                return c

            lax.fori_loop(0, CH // 16, grp, 0)
            pltpu.sync_copy(val_v, e_acc.at[ridx_v], add=True)
            return carry

        lax.fori_loop(0, NCHK, chunk, 0)
        plsc.subcore_barrier()
        pltpu.sync_copy(e_acc.at[pl.ds(rbase, RPT)],
                        ep_hbm.at[cid, pl.ds(rbase, RPT)])

    return sc_kernel(a_node, a_edge, src, dst, einf_flat)


def kernel(x, edge_index, edge_inform, W, att_l, bias):
    N, D_IN = x.shape
    E, DE = edge_inform.shape
    D_OUT = W.shape[0]

    src = edge_index[0]
    dst = edge_index[1]
    att2 = att_l.reshape(D_OUT, 1)
    WxT = W[:, :D_IN].T
    WeT = W[:, D_IN:].T
    wea2 = jnp.dot(WeT, att2)
    bias2 = bias.reshape(1, D_OUT)

    xW, a_node2 = _t1a(x, WxT, att2)
    a_edge2 = _t1b(edge_inform, wea2)

    E_pad = ((E + NW * CH - 1) // (NW * CH)) * (NW * CH)
    pad = E_pad - E
    a_node = a_node2.reshape(N)
    a_edge = jnp.pad(a_edge2.reshape(E), (0, pad), constant_values=-1e30)
    src_p = jnp.pad(src, (0, pad))
    dst_p = jnp.pad(dst, (0, pad))
    einf_flat = jnp.pad(edge_inform, ((0, pad), (0, 0))).reshape(E_pad * DE)

    out_p = _sc_main(a_node, a_edge, src_p, dst_p, xW)
    a_node_dep = a_node + 0.0 * out_p[0, 0, 0]
    e_p4 = _sc_edge(a_node_dep, a_edge, src_p, dst_p, einf_flat, DE)
    e_p = e_p4.reshape(NC, e_p4.shape[1] * 4, 32)
    return _t2(out_p, e_p, WeT, bias2, N)


# speedup vs baseline: 6.5598x; 1.0834x over previous
import functools

import jax
import jax.numpy as jnp
from jax import lax
from jax.experimental import pallas as pl
from jax.experimental.pallas import tpu as pltpu
from jax.experimental.pallas import tpu_sc as plsc

NC = 2
NS = 16
NW = NC * NS
CH = 128


def _t1a(x, WxT, att2):
    N, D = x.shape
    BN = 1000

    def body(x_ref, w_ref, a_ref, xw_ref, an_ref):
        xw = jnp.dot(x_ref[...], w_ref[...], preferred_element_type=jnp.float32)
        xw_ref[...] = xw
        an_ref[...] = jnp.dot(xw, a_ref[...], preferred_element_type=jnp.float32)

    return pl.pallas_call(
        body,
        grid=(N // BN,),
        in_specs=[
            pl.BlockSpec((BN, D), lambda i: (i, 0)),
            pl.BlockSpec((D, D), lambda i: (0, 0)),
            pl.BlockSpec((D, 1), lambda i: (0, 0)),
        ],
        out_specs=[
            pl.BlockSpec((BN, D), lambda i: (i, 0)),
            pl.BlockSpec((BN, 1), lambda i: (i, 0)),
        ],
        out_shape=[
            jax.ShapeDtypeStruct((N, D), jnp.float32),
            jax.ShapeDtypeStruct((N, 1), jnp.float32),
        ],
    )(x, WxT, att2)


def _t1b(einf, wea2):
    E, DE = einf.shape
    BE = 6400

    def body(e_ref, w_ref, o_ref):
        o_ref[...] = jnp.dot(e_ref[...], w_ref[...], preferred_element_type=jnp.float32)

    return pl.pallas_call(
        body,
        grid=(E // BE,),
        in_specs=[
            pl.BlockSpec((BE, DE), lambda i: (i, 0)),
            pl.BlockSpec((DE, 1), lambda i: (0, 0)),
        ],
        out_specs=pl.BlockSpec((BE, 1), lambda i: (i, 0)),
        out_shape=jax.ShapeDtypeStruct((E, 1), jnp.float32),
    )(einf, wea2)


def _t2(out_p, e_p, WeT, bias2, N):
    _, NPo, D = out_p.shape
    DE = WeT.shape[0]
    BN = 1000

    def body(op_ref, ep_ref, w_ref, b_ref, o_ref):
        s = op_ref[0] + op_ref[1]
        ef = ep_ref[0] + ep_ref[1]
        e16 = ef[:, 0:DE]
        denom = ef[:, DE:DE + 1]
        num = s + jnp.dot(e16, w_ref[...], preferred_element_type=jnp.float32)
        o_ref[...] = num / (denom + 1e-16) + b_ref[...]

    return pl.pallas_call(
        body,
        grid=(N // BN,),
        in_specs=[
            pl.BlockSpec((2, BN, D), lambda i: (0, i, 0)),
            pl.BlockSpec((2, BN, 32), lambda i: (0, i, 0)),
            pl.BlockSpec((DE, D), lambda i: (0, 0)),
            pl.BlockSpec((1, D), lambda i: (0, 0)),
        ],
        out_specs=pl.BlockSpec((BN, D), lambda i: (i, 0)),
        out_shape=jax.ShapeDtypeStruct((N, D), jnp.float32),
    )(out_p, e_p, WeT, bias2)


def _sc_main(a_node, a_edge, src, dst, xW):
    N = a_node.shape[0]
    D = xW.shape[1]
    E_pad = a_edge.shape[0]
    EPW = E_pad // NW
    NCHK = EPW // CH
    assert NCHK % 2 == 0
    RPT = ((N // NS) + 7) // 8 * 8
    LAST = N - (NS - 1) * RPT
    assert LAST % 8 == 0 and 0 < LAST <= RPT and N % 8 == 0

    mesh = plsc.VectorSubcoreMesh(core_axis_name="c", subcore_axis_name="s",
                                  num_cores=NC, num_subcores=NS)

    @functools.partial(
        pl.kernel,
        out_type=jax.ShapeDtypeStruct((NC, N, D), jnp.float32),
        mesh=mesh,
        compiler_params=pltpu.CompilerParams(needs_layout_passes=False),
        scratch_types=[
            pltpu.VMEM((N,), jnp.float32),
            pltpu.VMEM((CH,), jnp.int32),
            pltpu.VMEM((CH,), jnp.int32),
            pltpu.VMEM((CH,), jnp.float32),
            pltpu.VMEM((CH, D), jnp.float32),
            pltpu.VMEM((CH,), jnp.int32),
            pltpu.VMEM((CH,), jnp.int32),
            pltpu.VMEM((CH,), jnp.float32),
            pltpu.VMEM((CH, D), jnp.float32),
            pltpu.VMEM((CH,), jnp.float32),
            pltpu.VMEM_SHARED((N, D), jnp.float32),
            pltpu.SemaphoreType.DMA,
            pltpu.SemaphoreType.DMA,
            pltpu.SemaphoreType.DMA,
            pltpu.SemaphoreType.DMA,
        ],
    )
    def sc_kernel(an_hbm, ae_hbm, src_hbm, dst_hbm, xw_hbm, outp_hbm,
                  an_v, src0, dst0, ae0, rows0, src1, dst1, ae1, rows1,
                  ex_v, out_acc, lsem0, lsem1, gsem0, gsem1):
        cid = lax.axis_index("c")
        sid = lax.axis_index("s")
        wid = sid * NC + cid
        notlast = sid != NS - 1
        bufs = ((src0, dst0, ae0, rows0, lsem0, gsem0),
                (src1, dst1, ae1, rows1, lsem1, gsem1))

        def cbase(ci):
            return wid * EPW + jnp.minimum(ci, NCHK - 1) * CH

        def lin_issue(ci, b):
            base = cbase(ci)
            s_v, d_v, a_v, _, lsem, _ = bufs[b]
            pltpu.async_copy(src_hbm.at[pl.ds(base, CH)], s_v, lsem)
            pltpu.async_copy(dst_hbm.at[pl.ds(base, CH)], d_v, lsem)
            pltpu.async_copy(ae_hbm.at[pl.ds(base, CH)], a_v, lsem)

        def lin_wait(b):
            s_v, d_v, a_v, _, lsem, _ = bufs[b]
            pltpu.make_async_copy(src_hbm.at[pl.ds(0, CH)], s_v, lsem).wait()
            pltpu.make_async_copy(dst_hbm.at[pl.ds(0, CH)], d_v, lsem).wait()
            pltpu.make_async_copy(ae_hbm.at[pl.ds(0, CH)], a_v, lsem).wait()

        def gather_issue(b):
            s_v, _, _, r_v, _, gsem = bufs[b]
            pltpu.async_copy(xw_hbm.at[s_v], r_v, gsem)

        def gather_wait(b):
            s_v, _, _, r_v, _, gsem = bufs[b]
            pltpu.make_async_copy(xw_hbm.at[s_v], r_v, gsem).wait()

        def compute(b):
            s_v, d_v, a_v, r_v, _, _ = bufs[b]

            def grp(g, c):
                s16 = s_v[pl.ds(g * 16, 16)]
                an = plsc.load_gather(an_v, [s16])
                al = an + a_v[pl.ds(g * 16, 16)]
                al = jnp.maximum(al, al * 0.01)
                ex_v[pl.ds(g * 16, 16)] = jnp.exp(al)
                return c

            lax.fori_loop(0, CH // 16, grp, 0)

            def scale(g, c):
                ex16 = ex_v[pl.ds(g * 16, 16)]
                for l in range(16):
                    exs = ex16[l]
                    i = g * 16 + l
                    for j in range(D // 16):
                        sl = pl.ds(j * 16, 16)
                        r_v[i, sl] = r_v[i, sl] * exs
                return c

            lax.fori_loop(0, CH // 16, scale, 0)
            pltpu.sync_copy(r_v, out_acc.at[d_v], add=True)

        zero16 = jnp.zeros((16,), jnp.float32)

        def zrow(r, carry):
            for j in range(D // 16):
                rows0[r, pl.ds(j * 16, 16)] = zero16
            return carry

        lax.fori_loop(0, CH, zrow, 0)
        rbase = sid * RPT
        for k in range(LAST // CH):
            pltpu.sync_copy(rows0, out_acc.at[pl.ds(rbase + k * CH, CH)])

        @pl.when(notlast)
        def _():
            for k in range(LAST // CH, RPT // CH):
                pltpu.sync_copy(rows0, out_acc.at[pl.ds(rbase + k * CH, CH)])
            if RPT % CH:
                pltpu.sync_copy(rows0.at[pl.ds(0, RPT % CH)],
                                out_acc.at[pl.ds(rbase + (RPT // CH) * CH,
                                                 RPT % CH)])

        if LAST % CH:
            @pl.when(jnp.logical_not(notlast))
            def _():
                pltpu.sync_copy(rows0.at[pl.ds(0, LAST % CH)],
                                out_acc.at[pl.ds(rbase + (LAST // CH) * CH,
                                                 LAST % CH)])

        pltpu.sync_copy(an_hbm, an_v)
        plsc.subcore_barrier()

        lin_issue(0, 0)
        lin_issue(1, 1)
        lin_wait(0)
        gather_issue(0)

        def piter(k, carry):
            ci = 2 * k
            lin_wait(1)
            gather_wait(0)
            gather_issue(1)
            compute(0)
            lin_issue(ci + 2, 0)
            lin_wait(0)
            gather_wait(1)
            gather_issue(0)
            compute(1)
            lin_issue(ci + 3, 1)
            return carry

        lax.fori_loop(0, NCHK // 2, piter, 0)
        gather_wait(0)
        lin_wait(1)
        plsc.subcore_barrier()

        @pl.when(notlast)
        def _():
            pltpu.sync_copy(out_acc.at[pl.ds(rbase, RPT)],
                            outp_hbm.at[cid, pl.ds(rbase, RPT)])

        @pl.when(jnp.logical_not(notlast))
        def _():
            pltpu.sync_copy(out_acc.at[pl.ds(rbase, LAST)],
                            outp_hbm.at[cid, pl.ds(rbase, LAST)])

    return sc_kernel(a_node, a_edge, src, dst, xW)


def _sc_edge(a_node, a_edge, src, dst, einf_flat, DE):
    N = a_node.shape[0]
    E_pad = a_edge.shape[0]
    EPW = E_pad // NW
    NCHK = EPW // CH
    assert NCHK % 2 == 0
    NPe = ((N + NS * 32 - 1) // (NS * 32)) * (NS * 32)
    NP4 = NPe // 4
    RPT = NP4 // NS

    mesh = plsc.VectorSubcoreMesh(core_axis_name="c", subcore_axis_name="s",
                                  num_cores=NC, num_subcores=NS)

    @functools.partial(
        pl.kernel,
        out_type=jax.ShapeDtypeStruct((NC, NP4, 128), jnp.float32),
        mesh=mesh,
        compiler_params=pltpu.CompilerParams(needs_layout_passes=False),
        scratch_types=[
            pltpu.VMEM((N,), jnp.float32),
            pltpu.VMEM((CH,), jnp.int32),
            pltpu.VMEM((CH,), jnp.int32),
            pltpu.VMEM((CH,), jnp.float32),
            pltpu.VMEM((CH * 16,), jnp.float32),
            pltpu.VMEM((CH,), jnp.int32),
            pltpu.VMEM((CH,), jnp.int32),
            pltpu.VMEM((CH,), jnp.float32),
            pltpu.VMEM((CH * 16,), jnp.float32),
            pltpu.VMEM((CH,), jnp.int32),
            pltpu.VMEM((CH, 128), jnp.float32),
            pltpu.VMEM_SHARED((NP4, 128), jnp.float32),
            pltpu.SemaphoreType.DMA,
            pltpu.SemaphoreType.DMA,
        ],
    )
    def sc_kernel(an_hbm, ae_hbm, src_hbm, dst_hbm, einf_hbm, ep_hbm,
                  an_v, src0, dst0, ae0, ei0, src1, dst1, ae1, ei1,
                  ridx_v, val_v, e_acc, lsem0, lsem1):
        cid = lax.axis_index("c")
        sid = lax.axis_index("s")
        wid = sid * NC + cid
        bufs = ((src0, dst0, ae0, ei0, lsem0),
                (src1, dst1, ae1, ei1, lsem1))

        def cbase(ci):
            return wid * EPW + jnp.minimum(ci, NCHK - 1) * CH

        def lin_issue(ci, b):
            base = cbase(ci)
            s_v, d_v, a_v, e_v, lsem = bufs[b]
            pltpu.async_copy(src_hbm.at[pl.ds(base, CH)], s_v, lsem)
            pltpu.async_copy(dst_hbm.at[pl.ds(base, CH)], d_v, lsem)
            pltpu.async_copy(ae_hbm.at[pl.ds(base, CH)], a_v, lsem)
            pltpu.async_copy(einf_hbm.at[pl.ds(base * DE, CH * DE)], e_v, lsem)

        def lin_wait(b):
            s_v, d_v, a_v, e_v, lsem = bufs[b]
            pltpu.make_async_copy(src_hbm.at[pl.ds(0, CH)], s_v, lsem).wait()
            pltpu.make_async_copy(dst_hbm.at[pl.ds(0, CH)], d_v, lsem).wait()
            pltpu.make_async_copy(ae_hbm.at[pl.ds(0, CH)], a_v, lsem).wait()
            pltpu.make_async_copy(einf_hbm.at[pl.ds(0, CH * DE)], e_v,
                                  lsem).wait()

        zero16 = jnp.zeros((16,), jnp.float32)
        lane0 = lax.iota(jnp.int32, 16) == 0

        def compute(b):
            s_v, d_v, a_v, e_v, _ = bufs[b]

            def grp(g, c):
                s16 = s_v[pl.ds(g * 16, 16)]
                d16 = d_v[pl.ds(g * 16, 16)]
                ridx_v[pl.ds(g * 16, 16)] = lax.shift_right_logical(d16, 2)
                dmod = lax.bitwise_and(d16, 3)
                an = plsc.load_gather(an_v, [s16])
                al = an + a_v[pl.ds(g * 16, 16)]
                al = jnp.maximum(al, al * 0.01)
                ex16 = jnp.exp(al)
                for l in range(16):
                    exs = ex16[l]
                    q = dmod[l]
                    i = g * 16 + l
                    for j in range(8):
                        val_v[i, pl.ds(j * 16, 16)] = zero16
                    val_v[i, pl.ds(q * 32, 16)] = \
                        e_v[pl.ds(i * DE, 16)] * exs
                    val_v[i, pl.ds(q * 32 + 16, 16)] = \
                        jnp.where(lane0, exs, 0.0)
                return c

            lax.fori_loop(0, CH // 16, grp, 0)
            pltpu.sync_copy(val_v, e_acc.at[ridx_v], add=True)

        def zrow(r, carry):
            for j in range(8):
                val_v[r, pl.ds(j * 16, 16)] = zero16
            return carry

        lax.fori_loop(0, CH, zrow, 0)
        rbase = sid * RPT
        for k in range(RPT // CH):
            pltpu.sync_copy(val_v, e_acc.at[pl.ds(rbase + k * CH, CH)])
        zrem = RPT - (RPT // CH) * CH
        if zrem:
            pltpu.sync_copy(val_v.at[pl.ds(0, zrem)],
                            e_acc.at[pl.ds(rbase + (RPT // CH) * CH, zrem)])
        pltpu.sync_copy(an_hbm, an_v)
        plsc.subcore_barrier()

        lin_issue(0, 0)
        lin_issue(1, 1)

        def piter(k, carry):
            ci = 2 * k
            lin_wait(0)
            compute(0)
            lin_issue(ci + 2, 0)
            lin_wait(1)
            compute(1)
            lin_issue(ci + 3, 1)
            return carry

        lax.fori_loop(0, NCHK // 2, piter, 0)
        lin_wait(0)
        lin_wait(1)
        plsc.subcore_barrier()
        pltpu.sync_copy(e_acc.at[pl.ds(rbase, RPT)],
                        ep_hbm.at[cid, pl.ds(rbase, RPT)])

    return sc_kernel(a_node, a_edge, src, dst, einf_flat)


def kernel(x, edge_index, edge_inform, W, att_l, bias):
    N, D_IN = x.shape
    E, DE = edge_inform.shape
    D_OUT = W.shape[0]

    src = edge_index[0]
    dst = edge_index[1]
    att2 = att_l.reshape(D_OUT, 1)
    WxT = W[:, :D_IN].T
    WeT = W[:, D_IN:].T
    wea2 = jnp.dot(WeT, att2)
    bias2 = bias.reshape(1, D_OUT)

    xW, a_node2 = _t1a(x, WxT, att2)
    a_edge2 = _t1b(edge_inform, wea2)

    E_pad = ((E + 2 * NW * CH - 1) // (2 * NW * CH)) * (2 * NW * CH)
    pad = E_pad - E
    a_node = a_node2.reshape(N)
    a_edge = jnp.pad(a_edge2.reshape(E), (0, pad), constant_values=-1e30)
    src_p = jnp.pad(src, (0, pad))
    dst_p = jnp.pad(dst, (0, pad))
    einf_flat = jnp.pad(edge_inform, ((0, pad), (0, 0))).reshape(E_pad * DE)

    out_p = _sc_main(a_node, a_edge, src_p, dst_p, xW)
    a_node_dep = a_node + 0.0 * out_p[0, 0, 0]
    e_p4 = _sc_edge(a_node_dep, a_edge, src_p, dst_p, einf_flat, DE)
    e_p = e_p4.reshape(NC, e_p4.shape[1] * 4, 32)
    return _t2(out_p, e_p, WeT, bias2, N)


# speedup vs baseline: 10.9478x; 1.6689x over previous
import functools

import jax
import jax.numpy as jnp
from jax import lax
from jax.experimental import pallas as pl
from jax.experimental.pallas import tpu as pltpu
from jax.experimental.pallas import tpu_sc as plsc

NC = 2
NS = 16
NW = NC * NS
CH = 128


def _t1a(x, WxT, att2):
    N, D = x.shape
    BN = 1000

    def body(x_ref, w_ref, a_ref, xw_ref, an_ref):
        xw = jnp.dot(x_ref[...], w_ref[...], preferred_element_type=jnp.float32)
        xw_ref[...] = xw
        an_ref[...] = jnp.dot(xw, a_ref[...], preferred_element_type=jnp.float32)

    return pl.pallas_call(
        body,
        grid=(N // BN,),
        in_specs=[
            pl.BlockSpec((BN, D), lambda i: (i, 0)),
            pl.BlockSpec((D, D), lambda i: (0, 0)),
            pl.BlockSpec((D, 1), lambda i: (0, 0)),
        ],
        out_specs=[
            pl.BlockSpec((BN, D), lambda i: (i, 0)),
            pl.BlockSpec((BN, 1), lambda i: (i, 0)),
        ],
        out_shape=[
            jax.ShapeDtypeStruct((N, D), jnp.float32),
            jax.ShapeDtypeStruct((N, 1), jnp.float32),
        ],
    )(x, WxT, att2)


def _t1b(einf, wea2):
    E, DE = einf.shape
    BE = 6400

    def body(e_ref, w_ref, o_ref):
        o_ref[...] = jnp.dot(e_ref[...], w_ref[...], preferred_element_type=jnp.float32)

    return pl.pallas_call(
        body,
        grid=(E // BE,),
        in_specs=[
            pl.BlockSpec((BE, DE), lambda i: (i, 0)),
            pl.BlockSpec((DE, 1), lambda i: (0, 0)),
        ],
        out_specs=pl.BlockSpec((BE, 1), lambda i: (i, 0)),
        out_shape=jax.ShapeDtypeStruct((E, 1), jnp.float32),
    )(einf, wea2)


def _t2(out_p, e_p, WeT, bias2, N):
    _, NPo, D = out_p.shape
    DE = WeT.shape[0]
    BN = 1000

    def body(op_ref, ep_ref, w_ref, b_ref, o_ref):
        s = op_ref[0] + op_ref[1]
        ef = ep_ref[0] + ep_ref[1]
        e16 = ef[:, 0:DE]
        denom = ef[:, DE:DE + 1]
        num = s + jnp.dot(e16, w_ref[...], preferred_element_type=jnp.float32)
        o_ref[...] = num / (denom + 1e-16) + b_ref[...]

    return pl.pallas_call(
        body,
        grid=(N // BN,),
        in_specs=[
            pl.BlockSpec((2, BN, D), lambda i: (0, i, 0)),
            pl.BlockSpec((2, BN, 32), lambda i: (0, i, 0)),
            pl.BlockSpec((DE, D), lambda i: (0, 0)),
            pl.BlockSpec((1, D), lambda i: (0, 0)),
        ],
        out_specs=pl.BlockSpec((BN, D), lambda i: (i, 0)),
        out_shape=jax.ShapeDtypeStruct((N, D), jnp.float32),
    )(out_p, e_p, WeT, bias2)


def _sc_main(a_node, a_edge, src, dst, xW):
    N = a_node.shape[0]
    D = xW.shape[1]
    E = a_edge.shape[0]
    GCH = E // CH
    NR = GCH // NW
    NRP = NR if NR % 2 == 0 else NR - 1
    TAILR = list(range(NRP, NR))
    TAILW = GCH - NR * NW
    assert E % CH == 0
    RPT = ((N // NS) + 7) // 8 * 8
    LAST = N - (NS - 1) * RPT
    assert LAST % 8 == 0 and 0 < LAST <= RPT and N % 8 == 0

    mesh = plsc.VectorSubcoreMesh(core_axis_name="c", subcore_axis_name="s",
                                  num_cores=NC, num_subcores=NS)

    @functools.partial(
        pl.kernel,
        out_type=jax.ShapeDtypeStruct((NC, N, D), jnp.float32),
        mesh=mesh,
        compiler_params=pltpu.CompilerParams(needs_layout_passes=False),
        scratch_types=[
            pltpu.VMEM((N,), jnp.float32),
            pltpu.VMEM((CH,), jnp.int32),
            pltpu.VMEM((CH,), jnp.int32),
            pltpu.VMEM((CH,), jnp.float32),
            pltpu.VMEM((CH, D), jnp.float32),
            pltpu.VMEM((CH,), jnp.int32),
            pltpu.VMEM((CH,), jnp.int32),
            pltpu.VMEM((CH,), jnp.float32),
            pltpu.VMEM((CH, D), jnp.float32),
            pltpu.VMEM((CH,), jnp.float32),
            pltpu.VMEM_SHARED((N, D), jnp.float32),
            pltpu.SemaphoreType.DMA,
            pltpu.SemaphoreType.DMA,
            pltpu.SemaphoreType.DMA,
            pltpu.SemaphoreType.DMA,
        ],
    )
    def sc_kernel(an_hbm, ae_hbm, src_hbm, dst_hbm, xw_hbm, outp_hbm,
                  an_v, src0, dst0, ae0, rows0, src1, dst1, ae1, rows1,
                  ex_v, out_acc, lsem0, lsem1, gsem0, gsem1):
        cid = lax.axis_index("c")
        sid = lax.axis_index("s")
        wid = sid * NC + cid
        notlast = sid != NS - 1
        bufs = ((src0, dst0, ae0, rows0, lsem0, gsem0),
                (src1, dst1, ae1, rows1, lsem1, gsem1))

        def cbase(r):
            return jnp.minimum(r * NW + wid, GCH - 1) * CH

        def lin_issue(r, b):
            base = cbase(r)
            s_v, d_v, a_v, _, lsem, _ = bufs[b]
            pltpu.async_copy(src_hbm.at[pl.ds(base, CH)], s_v, lsem)
            pltpu.async_copy(dst_hbm.at[pl.ds(base, CH)], d_v, lsem)
            pltpu.async_copy(ae_hbm.at[pl.ds(base, CH)], a_v, lsem)

        def lin_wait(b):
            s_v, d_v, a_v, _, lsem, _ = bufs[b]
            pltpu.make_async_copy(src_hbm.at[pl.ds(0, CH)], s_v, lsem).wait()
            pltpu.make_async_copy(dst_hbm.at[pl.ds(0, CH)], d_v, lsem).wait()
            pltpu.make_async_copy(ae_hbm.at[pl.ds(0, CH)], a_v, lsem).wait()

        def gather_issue(b):
            s_v, _, _, r_v, _, gsem = bufs[b]
            pltpu.async_copy(xw_hbm.at[s_v], r_v, gsem)

        def gather_wait(b):
            s_v, _, _, r_v, _, gsem = bufs[b]
            pltpu.make_async_copy(xw_hbm.at[s_v], r_v, gsem).wait()

        def compute(b):
            s_v, d_v, a_v, r_v, _, _ = bufs[b]

            def grp(g, c):
                s16 = s_v[pl.ds(g * 16, 16)]
                an = plsc.load_gather(an_v, [s16])
                al = an + a_v[pl.ds(g * 16, 16)]
                al = jnp.maximum(al, al * 0.01)
                ex_v[pl.ds(g * 16, 16)] = jnp.exp(al)
                return c

            lax.fori_loop(0, CH // 16, grp, 0)

            def scale(g, c):
                ex16 = ex_v[pl.ds(g * 16, 16)]
                for l in range(16):
                    exs = ex16[l]
                    i = g * 16 + l
                    for j in range(D // 16):
                        sl = pl.ds(j * 16, 16)
                        r_v[i, sl] = r_v[i, sl] * exs
                return c

            lax.fori_loop(0, CH // 16, scale, 0)
            pltpu.sync_copy(r_v, out_acc.at[d_v], add=True)

        zero16 = jnp.zeros((16,), jnp.float32)

        def zrow(r, carry):
            for j in range(D // 16):
                rows0[r, pl.ds(j * 16, 16)] = zero16
            return carry

        lax.fori_loop(0, CH, zrow, 0)
        rbase = sid * RPT
        for k in range(LAST // CH):
            pltpu.sync_copy(rows0, out_acc.at[pl.ds(rbase + k * CH, CH)])

        @pl.when(notlast)
        def _():
            for k in range(LAST // CH, RPT // CH):
                pltpu.sync_copy(rows0, out_acc.at[pl.ds(rbase + k * CH, CH)])
            if RPT % CH:
                pltpu.sync_copy(rows0.at[pl.ds(0, RPT % CH)],
                                out_acc.at[pl.ds(rbase + (RPT // CH) * CH,
                                                 RPT % CH)])

        if LAST % CH:
            @pl.when(jnp.logical_not(notlast))
            def _():
                pltpu.sync_copy(rows0.at[pl.ds(0, LAST % CH)],
                                out_acc.at[pl.ds(rbase + (LAST // CH) * CH,
                                                 LAST % CH)])

        pltpu.sync_copy(an_hbm, an_v)
        plsc.subcore_barrier()

        lin_issue(0, 0)
        lin_issue(1, 1)
        lin_wait(0)
        gather_issue(0)

        def piter(k, carry):
            r = 2 * k
            lin_wait(1)
            gather_wait(0)
            gather_issue(1)
            compute(0)
            lin_issue(r + 2, 0)
            lin_wait(0)
            gather_wait(1)
            gather_issue(0)
            compute(1)
            lin_issue(r + 3, 1)
            return carry

        lax.fori_loop(0, NRP // 2, piter, 0)
        gather_wait(0)
        lin_wait(1)

        for r in TAILR:
            lin_issue(r, 0)
            lin_wait(0)
            gather_issue(0)
            gather_wait(0)
            compute(0)
        if TAILW:
            @pl.when(wid < TAILW)
            def _():
                lin_issue(NR, 0)
                lin_wait(0)
                gather_issue(0)
                gather_wait(0)
                compute(0)

        plsc.subcore_barrier()

        @pl.when(notlast)
        def _():
            pltpu.sync_copy(out_acc.at[pl.ds(rbase, RPT)],
                            outp_hbm.at[cid, pl.ds(rbase, RPT)])

        @pl.when(jnp.logical_not(notlast))
        def _():
            pltpu.sync_copy(out_acc.at[pl.ds(rbase, LAST)],
                            outp_hbm.at[cid, pl.ds(rbase, LAST)])

    return sc_kernel(a_node, a_edge, src, dst, xW)


def _sc_edge(a_node, a_edge, src, dst, einf_pk, DE):
    N = a_node.shape[0]
    E = a_edge.shape[0]
    GCH = E // CH
    NR = GCH // NW
    NRP = NR if NR % 2 == 0 else NR - 1
    TAILR = list(range(NRP, NR))
    TAILW = GCH - NR * NW
    assert E % CH == 0
    EIR = CH // 8
    NPe = ((N + NS * 32 - 1) // (NS * 32)) * (NS * 32)
    NP4 = NPe // 4
    RPT = NP4 // NS

    mesh = plsc.VectorSubcoreMesh(core_axis_name="c", subcore_axis_name="s",
                                  num_cores=NC, num_subcores=NS)

    @functools.partial(
        pl.kernel,
        out_type=jax.ShapeDtypeStruct((NC, NP4, 128), jnp.float32),
        mesh=mesh,
        compiler_params=pltpu.CompilerParams(needs_layout_passes=False),
        scratch_types=[
            pltpu.VMEM((N,), jnp.float32),
            pltpu.VMEM((CH,), jnp.int32),
            pltpu.VMEM((CH,), jnp.int32),
            pltpu.VMEM((CH,), jnp.float32),
            pltpu.VMEM((EIR, 128), jnp.float32),
            pltpu.VMEM((CH,), jnp.int32),
            pltpu.VMEM((CH,), jnp.int32),
            pltpu.VMEM((CH,), jnp.float32),
            pltpu.VMEM((EIR, 128), jnp.float32),
            pltpu.VMEM((CH,), jnp.int32),
            pltpu.VMEM((CH, 128), jnp.float32),
            pltpu.VMEM_SHARED((NP4, 128), jnp.float32),
            pltpu.SemaphoreType.DMA,
            pltpu.SemaphoreType.DMA,
        ],
    )
    def sc_kernel(an_hbm, ae_hbm, src_hbm, dst_hbm, einf_hbm, ep_hbm,
                  an_v, src0, dst0, ae0, ei0, src1, dst1, ae1, ei1,
                  ridx_v, val_v, e_acc, lsem0, lsem1):
        cid = lax.axis_index("c")
        sid = lax.axis_index("s")
        wid = sid * NC + cid
        bufs = ((src0, dst0, ae0, ei0, lsem0),
                (src1, dst1, ae1, ei1, lsem1))

        def gidx(r):
            return jnp.minimum(r * NW + wid, GCH - 1)

        def lin_issue(r, b):
            gi = gidx(r)
            base = gi * CH
            s_v, d_v, a_v, e_v, lsem = bufs[b]
            pltpu.async_copy(src_hbm.at[pl.ds(base, CH)], s_v, lsem)
            pltpu.async_copy(dst_hbm.at[pl.ds(base, CH)], d_v, lsem)
            pltpu.async_copy(ae_hbm.at[pl.ds(base, CH)], a_v, lsem)
            pltpu.async_copy(einf_hbm.at[pl.ds(gi * EIR, EIR)], e_v, lsem)

        def lin_wait(b):
            s_v, d_v, a_v, e_v, lsem = bufs[b]
            pltpu.make_async_copy(src_hbm.at[pl.ds(0, CH)], s_v, lsem).wait()
            pltpu.make_async_copy(dst_hbm.at[pl.ds(0, CH)], d_v, lsem).wait()
            pltpu.make_async_copy(ae_hbm.at[pl.ds(0, CH)], a_v, lsem).wait()
            pltpu.make_async_copy(einf_hbm.at[pl.ds(0, EIR)], e_v,
                                  lsem).wait()

        zero16 = jnp.zeros((16,), jnp.float32)
        lane0 = lax.iota(jnp.int32, 16) == 0

        def compute(b):
            s_v, d_v, a_v, e_v, _ = bufs[b]

            def grp(g, c):
                s16 = s_v[pl.ds(g * 16, 16)]
                d16 = d_v[pl.ds(g * 16, 16)]
                ridx_v[pl.ds(g * 16, 16)] = lax.shift_right_logical(d16, 2)
                dmod = lax.bitwise_and(d16, 3)
                an = plsc.load_gather(an_v, [s16])
                al = an + a_v[pl.ds(g * 16, 16)]
                al = jnp.maximum(al, al * 0.01)
                ex16 = jnp.exp(al)
                for l in range(16):
                    exs = ex16[l]
                    q = dmod[l]
                    i = g * 16 + l
                    for j in range(8):
                        val_v[i, pl.ds(j * 16, 16)] = zero16
                    erow = e_v[2 * g + l // 8, pl.ds((l % 8) * 16, 16)]
                    val_v[i, pl.ds(q * 32, 16)] = erow * exs
                    val_v[i, pl.ds(q * 32 + 16, 16)] = \
                        jnp.where(lane0, exs, 0.0)
                return c

            lax.fori_loop(0, CH // 16, grp, 0)
            pltpu.sync_copy(val_v, e_acc.at[ridx_v], add=True)

        def zrow(r, carry):
            for j in range(8):
                val_v[r, pl.ds(j * 16, 16)] = zero16
            return carry

        lax.fori_loop(0, CH, zrow, 0)
        rbase = sid * RPT
        for k in range(RPT // CH):
            pltpu.sync_copy(val_v, e_acc.at[pl.ds(rbase + k * CH, CH)])
        zrem = RPT - (RPT // CH) * CH
        if zrem:
            pltpu.sync_copy(val_v.at[pl.ds(0, zrem)],
                            e_acc.at[pl.ds(rbase + (RPT // CH) * CH, zrem)])
        pltpu.sync_copy(an_hbm, an_v)
        plsc.subcore_barrier()

        lin_issue(0, 0)
        lin_issue(1, 1)

        def piter(k, carry):
            r = 2 * k
            lin_wait(0)
            compute(0)
            lin_issue(r + 2, 0)
            lin_wait(1)
            compute(1)
            lin_issue(r + 3, 1)
            return carry

        lax.fori_loop(0, NRP // 2, piter, 0)
        lin_wait(0)
        lin_wait(1)

        for r in TAILR:
            lin_issue(r, 0)
            lin_wait(0)
            compute(0)
        if TAILW:
            @pl.when(wid < TAILW)
            def _():
                lin_issue(NR, 0)
                lin_wait(0)
                compute(0)

        plsc.subcore_barrier()
        pltpu.sync_copy(e_acc.at[pl.ds(rbase, RPT)],
                        ep_hbm.at[cid, pl.ds(rbase, RPT)])

    return sc_kernel(a_node, a_edge, src, dst, einf_pk)


def kernel(x, edge_index, edge_inform, W, att_l, bias):
    N, D_IN = x.shape
    E, DE = edge_inform.shape
    D_OUT = W.shape[0]

    src = edge_index[0]
    dst = edge_index[1]
    att2 = att_l.reshape(D_OUT, 1)
    WxT = W[:, :D_IN].T
    WeT = W[:, D_IN:].T
    wea2 = jnp.dot(WeT, att2)
    bias2 = bias.reshape(1, D_OUT)

    xW, a_node2 = _t1a(x, WxT, att2)
    a_edge2 = _t1b(edge_inform, wea2)
    einf_pk = edge_inform.reshape(E // 8, 8 * DE)

    a_node = a_node2.reshape(N)
    a_edge = a_edge2.reshape(E)

    out_p = _sc_main(a_node, a_edge, src, dst, xW)
    a_node_dep = a_node + 0.0 * out_p[0, 0, 0]
    e_p4 = _sc_edge(a_node_dep, a_edge, src, dst, einf_pk, DE)
    e_p = e_p4.reshape(NC, e_p4.shape[1] * 4, 32)
    return _t2(out_p, e_p, WeT, bias2, N)
